# baseline (device time: 36637 ns/iter reference)
import jax
import jax.numpy as jnp
from jax import lax
from jax.experimental import pallas as pl
from jax.experimental.pallas import tpu as pltpu

N_DEV = 4
NCHUNK = 4


def kernel(x, Wp):
    b, h, w, c = x.shape
    cout = Wp.shape[1]
    hc = h // NCHUNK
    rows = hc * w
    n_global = (h * N_DEV) * w

    def body(
        x_hbm,
        wp_ref,
        out_hbm,
        x_vmem,
        outbuf,
        stats_ref,
        copy_sems,
        out_sems,
        send_sems,
        recv_sems,
    ):
        my = lax.axis_index("i")

        barrier_sem = pltpu.get_barrier_semaphore()
        for d in (1, 2, 3):
            pl.semaphore_signal(
                barrier_sem,
                inc=1,
                device_id=((my + d) % N_DEV,),
                device_id_type=pl.DeviceIdType.MESH,
            )

        copies = []
        for bi in range(b):
            for ci in range(NCHUNK):
                cp = pltpu.make_async_copy(
                    x_hbm.at[bi, pl.ds(ci * hc, hc), :, :],
                    x_vmem.at[bi, pl.ds(ci * hc, hc), :, :],
                    copy_sems.at[bi * NCHUNK + ci],
                )
                cp.start()
                copies.append(cp)
        for bi in range(b):
            s = None
            sq = None
            for ci in range(NCHUNK):
                copies[bi * NCHUNK + ci].wait()
                xb = x_vmem[bi, pl.ds(ci * hc, hc), :, :].reshape(rows, c)
                ps = jnp.sum(xb, axis=0, keepdims=True)
                psq = jnp.sum(xb * xb, axis=0, keepdims=True)
                s = ps if s is None else s + ps
                sq = psq if sq is None else sq + psq
            stats_ref[N_DEV - 1, 2 * bi : 2 * bi + 1, :] = s
            stats_ref[N_DEV - 1, 2 * bi + 1 : 2 * bi + 2, :] = sq

        pl.semaphore_wait(barrier_sem, N_DEV - 1)
        sends = []
        for d in (1, 2, 3):
            rdma = pltpu.make_async_remote_copy(
                src_ref=stats_ref.at[N_DEV - 1],
                dst_ref=stats_ref.at[d - 1],
                send_sem=send_sems.at[d - 1],
                recv_sem=recv_sems.at[d - 1],
                device_id=((my + d) % N_DEV,),
                device_id_type=pl.DeviceIdType.MESH,
            )
            rdma.start()
            sends.append(rdma)
        for d in (1, 2, 3):
            recv = pltpu.make_async_remote_copy(
                src_ref=stats_ref.at[N_DEV - 1],
                dst_ref=stats_ref.at[d - 1],
                send_sem=send_sems.at[d - 1],
                recv_sem=recv_sems.at[d - 1],
                device_id=((my + d) % N_DEV,),
                device_id_type=pl.DeviceIdType.MESH,
            )
            recv.wait_recv()
        for rdma in sends:
            rdma.wait_send()

        eps = 1e-5
        inv_n = 1.0 / float(n_global)
        means = []
        scales = []
        for bi in range(b):
            ssum = (
                stats_ref[0, 2 * bi : 2 * bi + 1, :]
                + stats_ref[1, 2 * bi : 2 * bi + 1, :]
                + stats_ref[2, 2 * bi : 2 * bi + 1, :]
                + stats_ref[3, 2 * bi : 2 * bi + 1, :]
            )
            ssq = (
                stats_ref[0, 2 * bi + 1 : 2 * bi + 2, :]
                + stats_ref[1, 2 * bi + 1 : 2 * bi + 2, :]
                + stats_ref[2, 2 * bi + 1 : 2 * bi + 2, :]
                + stats_ref[3, 2 * bi + 1 : 2 * bi + 2, :]
            )
            mean = ssum * inv_n
            var = ssq * inv_n - mean * mean
            means.append(mean)
            scales.append(lax.rsqrt(var + eps))

        wp = wp_ref[:, :]
        out_waits = [None, None]
        k = 0
        for bi in range(b):
            for ci in range(NCHUNK):
                slot = k % 2
                if out_waits[slot] is not None:
                    out_waits[slot].wait()
                xb = x_vmem[bi, pl.ds(ci * hc, hc), :, :].reshape(rows, c)
                hh = (xb - means[bi]) * scales[bi]
                a = hh * jax.nn.sigmoid(hh)
                res = jnp.dot(a, wp, preferred_element_type=jnp.float32)
                outbuf[slot] = res.reshape(hc, w, cout)
                cp = pltpu.make_async_copy(
                    outbuf.at[slot],
                    out_hbm.at[bi, pl.ds(ci * hc, hc), :, :],
                    out_sems.at[slot],
                )
                cp.start()
                out_waits[slot] = cp
                k += 1
        out_waits[0].wait()
        out_waits[1].wait()

    return pl.pallas_call(
        body,
        out_shape=jax.ShapeDtypeStruct((b, h, w, cout), jnp.float32),
        in_specs=[
            pl.BlockSpec(memory_space=pltpu.MemorySpace.HBM),
            pl.BlockSpec(memory_space=pltpu.MemorySpace.VMEM),
        ],
        out_specs=pl.BlockSpec(memory_space=pltpu.MemorySpace.HBM),
        scratch_shapes=[
            pltpu.VMEM((b, h, w, c), jnp.float32),
            pltpu.VMEM((2, hc, w, cout), jnp.float32),
            pltpu.VMEM((N_DEV, 2 * b, c), jnp.float32),
            pltpu.SemaphoreType.DMA((b * NCHUNK,)),
            pltpu.SemaphoreType.DMA((2,)),
            pltpu.SemaphoreType.DMA((N_DEV - 1,)),
            pltpu.SemaphoreType.DMA((N_DEV - 1,)),
        ],
        compiler_params=pltpu.CompilerParams(collective_id=0),
    )(x, Wp)


# device time: 35205 ns/iter; 1.0407x vs baseline; 1.0407x over previous
import jax
import jax.numpy as jnp
from jax import lax
from jax.experimental import pallas as pl
from jax.experimental.pallas import tpu as pltpu

N_DEV = 4
NCHUNK = 4


def kernel(x, Wp):
    b, h, w, c = x.shape
    cout = Wp.shape[1]
    hc = h // NCHUNK
    n_global = (h * N_DEV) * w
    xt = jnp.transpose(x, (0, 1, 3, 2))

    def body(
        x_hbm,
        wp_ref,
        out_hbm,
        x_vmem,
        outbuf,
        stats_ref,
        copy_sems,
        out_sems,
        send_sems,
        recv_sems,
    ):
        my = lax.axis_index("i")

        barrier_sem = pltpu.get_barrier_semaphore()
        for d in (1, 2, 3):
            pl.semaphore_signal(
                barrier_sem,
                inc=1,
                device_id=((my + d) % N_DEV,),
                device_id_type=pl.DeviceIdType.MESH,
            )

        copies = []
        for bi in range(b):
            for ci in range(NCHUNK):
                cp = pltpu.make_async_copy(
                    x_hbm.at[bi, pl.ds(ci * hc, hc), :, :],
                    x_vmem.at[bi, pl.ds(ci * hc, hc), :, :],
                    copy_sems.at[bi * NCHUNK + ci],
                )
                cp.start()
                copies.append(cp)
        for bi in range(b):
            s_cw = None
            sq_cw = None
            for ci in range(NCHUNK):
                copies[bi * NCHUNK + ci].wait()
                v = x_vmem[bi, pl.ds(ci * hc, hc), :, :]
                ps = jnp.sum(v, axis=0)
                psq = jnp.sum(v * v, axis=0)
                s_cw = ps if s_cw is None else s_cw + ps
                sq_cw = psq if sq_cw is None else sq_cw + psq
            stats_ref[N_DEV - 1, 2 * bi] = jnp.sum(s_cw, axis=1, keepdims=True)
            stats_ref[N_DEV - 1, 2 * bi + 1] = jnp.sum(
                sq_cw, axis=1, keepdims=True
            )

        pl.semaphore_wait(barrier_sem, N_DEV - 1)
        sends = []
        for d in (1, 2, 3):
            rdma = pltpu.make_async_remote_copy(
                src_ref=stats_ref.at[N_DEV - 1],
                dst_ref=stats_ref.at[d - 1],
                send_sem=send_sems.at[d - 1],
                recv_sem=recv_sems.at[d - 1],
                device_id=((my + d) % N_DEV,),
                device_id_type=pl.DeviceIdType.MESH,
            )
            rdma.start()
            sends.append(rdma)
        for d in (1, 2, 3):
            recv = pltpu.make_async_remote_copy(
                src_ref=stats_ref.at[N_DEV - 1],
                dst_ref=stats_ref.at[d - 1],
                send_sem=send_sems.at[d - 1],
                recv_sem=recv_sems.at[d - 1],
                device_id=((my + d) % N_DEV,),
                device_id_type=pl.DeviceIdType.MESH,
            )
            recv.wait_recv()
        for rdma in sends:
            rdma.wait_send()

        eps = 1e-5
        inv_n = 1.0 / float(n_global)
        means = []
        scales = []
        for bi in range(b):
            ssum = (
                stats_ref[0, 2 * bi]
                + stats_ref[1, 2 * bi]
                + stats_ref[2, 2 * bi]
                + stats_ref[3, 2 * bi]
            )
            ssq = (
                stats_ref[0, 2 * bi + 1]
                + stats_ref[1, 2 * bi + 1]
                + stats_ref[2, 2 * bi + 1]
                + stats_ref[3, 2 * bi + 1]
            )
            mean = ssum * inv_n
            var = ssq * inv_n - mean * mean
            means.append(mean[None, :, :])
            scales.append(lax.rsqrt(var + eps)[None, :, :])

        wp = wp_ref[:, :]
        out_waits = [None, None]
        k = 0
        for bi in range(b):
            for ci in range(NCHUNK):
                slot = k % 2
                if out_waits[slot] is not None:
                    out_waits[slot].wait()
                v = x_vmem[bi, pl.ds(ci * hc, hc), :, :]
                hh = (v - means[bi]) * scales[bi]
                a = hh * jax.nn.sigmoid(hh)
                res = lax.dot_general(
                    a,
                    wp,
                    dimension_numbers=(((1,), (0,)), ((), ())),
                    preferred_element_type=jnp.float32,
                )
                outbuf[slot] = res
                cp = pltpu.make_async_copy(
                    outbuf.at[slot],
                    out_hbm.at[bi, pl.ds(ci * hc, hc), :, :],
                    out_sems.at[slot],
                )
                cp.start()
                out_waits[slot] = cp
                k += 1
        out_waits[0].wait()
        out_waits[1].wait()

    return pl.pallas_call(
        body,
        out_shape=jax.ShapeDtypeStruct((b, h, w, cout), jnp.float32),
        in_specs=[
            pl.BlockSpec(memory_space=pltpu.MemorySpace.HBM),
            pl.BlockSpec(memory_space=pltpu.MemorySpace.VMEM),
        ],
        out_specs=pl.BlockSpec(memory_space=pltpu.MemorySpace.HBM),
        scratch_shapes=[
            pltpu.VMEM((b, h, c, w), jnp.float32),
            pltpu.VMEM((2, hc, w, cout), jnp.float32),
            pltpu.VMEM((N_DEV, 2 * b, c, 1), jnp.float32),
            pltpu.SemaphoreType.DMA((b * NCHUNK,)),
            pltpu.SemaphoreType.DMA((2,)),
            pltpu.SemaphoreType.DMA((N_DEV - 1,)),
            pltpu.SemaphoreType.DMA((N_DEV - 1,)),
        ],
        compiler_params=pltpu.CompilerParams(collective_id=0),
    )(xt, Wp)


# device time: 35018 ns/iter; 1.0462x vs baseline; 1.0053x over previous
import jax
import jax.numpy as jnp
from jax import lax
from jax.experimental import pallas as pl
from jax.experimental.pallas import tpu as pltpu

N_DEV = 4
NCHUNK = 4


def kernel(x, Wp):
    b, h, w, c = x.shape
    cout = Wp.shape[1]
    hc = h // NCHUNK
    n_global = (h * N_DEV) * w
    xt = jnp.transpose(x, (0, 1, 3, 2))

    def body(x_ref, wp_ref, out_ref, stats_ref, send_sems, recv_sems):
        my = lax.axis_index("i")

        barrier_sem = pltpu.get_barrier_semaphore()
        for d in (1, 2, 3):
            pl.semaphore_signal(
                barrier_sem,
                inc=1,
                device_id=((my + d) % N_DEV,),
                device_id_type=pl.DeviceIdType.MESH,
            )

        for bi in range(b):
            v = x_ref[bi]
            s_cw = jnp.sum(v, axis=0)
            sq_cw = jnp.sum(v * v, axis=0)
            stats_ref[N_DEV - 1, 2 * bi] = jnp.sum(s_cw, axis=1, keepdims=True)
            stats_ref[N_DEV - 1, 2 * bi + 1] = jnp.sum(
                sq_cw, axis=1, keepdims=True
            )

        pl.semaphore_wait(barrier_sem, N_DEV - 1)
        sends = []
        for d in (1, 2, 3):
            rdma = pltpu.make_async_remote_copy(
                src_ref=stats_ref.at[N_DEV - 1],
                dst_ref=stats_ref.at[d - 1],
                send_sem=send_sems.at[d - 1],
                recv_sem=recv_sems.at[d - 1],
                device_id=((my + d) % N_DEV,),
                device_id_type=pl.DeviceIdType.MESH,
            )
            rdma.start()
            sends.append(rdma)
        for d in (1, 2, 3):
            recv = pltpu.make_async_remote_copy(
                src_ref=stats_ref.at[N_DEV - 1],
                dst_ref=stats_ref.at[d - 1],
                send_sem=send_sems.at[d - 1],
                recv_sem=recv_sems.at[d - 1],
                device_id=((my + d) % N_DEV,),
                device_id_type=pl.DeviceIdType.MESH,
            )
            recv.wait_recv()
        for rdma in sends:
            rdma.wait_send()

        eps = 1e-5
        inv_n = 1.0 / float(n_global)
        means = []
        scales = []
        for bi in range(b):
            ssum = (
                stats_ref[0, 2 * bi]
                + stats_ref[1, 2 * bi]
                + stats_ref[2, 2 * bi]
                + stats_ref[3, 2 * bi]
            )
            ssq = (
                stats_ref[0, 2 * bi + 1]
                + stats_ref[1, 2 * bi + 1]
                + stats_ref[2, 2 * bi + 1]
                + stats_ref[3, 2 * bi + 1]
            )
            mean = ssum * inv_n
            var = ssq * inv_n - mean * mean
            means.append(mean[None, :, :])
            scales.append(lax.rsqrt(var + eps)[None, :, :])

        wp = wp_ref[:, :]
        for bi in range(b):
            for ci in range(NCHUNK):
                v = x_ref[bi, pl.ds(ci * hc, hc), :, :]
                hh = (v - means[bi]) * scales[bi]
                a = hh * jax.nn.sigmoid(hh)
                res = lax.dot_general(
                    a,
                    wp,
                    dimension_numbers=(((1,), (0,)), ((), ())),
                    preferred_element_type=jnp.float32,
                )
                out_ref[bi, pl.ds(ci * hc, hc), :, :] = res

    return pl.pallas_call(
        body,
        out_shape=jax.ShapeDtypeStruct((b, h, w, cout), jnp.float32),
        in_specs=[
            pl.BlockSpec(memory_space=pltpu.MemorySpace.VMEM),
            pl.BlockSpec(memory_space=pltpu.MemorySpace.VMEM),
        ],
        out_specs=pl.BlockSpec(memory_space=pltpu.MemorySpace.VMEM),
        scratch_shapes=[
            pltpu.VMEM((N_DEV, 2 * b, c, 1), jnp.float32),
            pltpu.SemaphoreType.DMA((N_DEV - 1,)),
            pltpu.SemaphoreType.DMA((N_DEV - 1,)),
        ],
        compiler_params=pltpu.CompilerParams(collective_id=0),
    )(xt, Wp)


# device time: 34996 ns/iter; 1.0469x vs baseline; 1.0006x over previous
import jax
import jax.numpy as jnp
from jax import lax
from jax.experimental import pallas as pl
from jax.experimental.pallas import tpu as pltpu

N_DEV = 4
NCHUNK = 4


def kernel(x, Wp):
    b, h, w, c = x.shape
    cout = Wp.shape[1]
    hc = h // NCHUNK
    n_global = (h * N_DEV) * w
    xt = jnp.transpose(x, (0, 1, 3, 2))

    def body(x_ref, wp_ref, out_ref, stats_ref, send_sems, recv_sems):
        my = lax.axis_index("i")

        barrier_sem = pltpu.get_barrier_semaphore()
        for d in (1, 2, 3):
            pl.semaphore_signal(
                barrier_sem,
                inc=1,
                device_id=((my + d) % N_DEV,),
                device_id_type=pl.DeviceIdType.MESH,
            )

        for bi in range(b):
            v = x_ref[bi]
            s_cw = jnp.sum(v, axis=0)
            sq_cw = jnp.sum(v * v, axis=0)
            stats_ref[N_DEV - 1, 2 * bi] = jnp.sum(s_cw, axis=1, keepdims=True)
            stats_ref[N_DEV - 1, 2 * bi + 1] = jnp.sum(
                sq_cw, axis=1, keepdims=True
            )

        pl.semaphore_wait(barrier_sem, N_DEV - 1)
        sends = []
        for d in (1, 2, 3):
            rdma = pltpu.make_async_remote_copy(
                src_ref=stats_ref.at[N_DEV - 1],
                dst_ref=stats_ref.at[d - 1],
                send_sem=send_sems.at[d - 1],
                recv_sem=recv_sems.at[d - 1],
                device_id=((my + d) % N_DEV,),
                device_id_type=pl.DeviceIdType.MESH,
            )
            rdma.start()
            sends.append(rdma)
        for d in (1, 2, 3):
            recv = pltpu.make_async_remote_copy(
                src_ref=stats_ref.at[N_DEV - 1],
                dst_ref=stats_ref.at[d - 1],
                send_sem=send_sems.at[d - 1],
                recv_sem=recv_sems.at[d - 1],
                device_id=((my + d) % N_DEV,),
                device_id_type=pl.DeviceIdType.MESH,
            )
            recv.wait_recv()
        for rdma in sends:
            rdma.wait_send()

        eps = 1e-5
        inv_n = 1.0 / float(n_global)
        means = []
        scales = []
        for bi in range(b):
            ssum = (
                stats_ref[0, 2 * bi]
                + stats_ref[1, 2 * bi]
                + stats_ref[2, 2 * bi]
                + stats_ref[3, 2 * bi]
            )
            ssq = (
                stats_ref[0, 2 * bi + 1]
                + stats_ref[1, 2 * bi + 1]
                + stats_ref[2, 2 * bi + 1]
                + stats_ref[3, 2 * bi + 1]
            )
            mean = ssum * inv_n
            var = ssq * inv_n - mean * mean
            means.append(mean[None, :, :])
            scales.append(lax.rsqrt(var + eps)[None, :, :])

        wp = wp_ref[:, :]
        for bi in range(b):
            for ci in range(NCHUNK):
                v = x_ref[bi, pl.ds(ci * hc, hc), :, :]
                hh = (v - means[bi]) * scales[bi]
                a = hh * jax.nn.sigmoid(hh)
                res = lax.dot_general(
                    a,
                    wp,
                    dimension_numbers=(((1,), (0,)), ((), ())),
                    preferred_element_type=jnp.float32,
                )
                out_ref[bi, pl.ds(ci * hc * w, hc * w), :] = res.reshape(
                    hc * w, cout
                )

    out = pl.pallas_call(
        body,
        out_shape=jax.ShapeDtypeStruct((b, h * w, cout), jnp.float32),
        in_specs=[
            pl.BlockSpec(memory_space=pltpu.MemorySpace.VMEM),
            pl.BlockSpec(memory_space=pltpu.MemorySpace.VMEM),
        ],
        out_specs=pl.BlockSpec(memory_space=pltpu.MemorySpace.VMEM),
        scratch_shapes=[
            pltpu.VMEM((N_DEV, 2 * b, c, 1), jnp.float32),
            pltpu.SemaphoreType.DMA((N_DEV - 1,)),
            pltpu.SemaphoreType.DMA((N_DEV - 1,)),
        ],
        compiler_params=pltpu.CompilerParams(collective_id=0),
    )(xt, Wp)
    return out.reshape(b, h, w, cout)


# device time: 21697 ns/iter; 1.6886x vs baseline; 1.6129x over previous
import jax
import jax.numpy as jnp
from jax import lax
from jax.experimental import pallas as pl
from jax.experimental.pallas import tpu as pltpu

N_DEV = 4
NCHUNK = 4


def kernel(x, Wp):
    b, h, w, c = x.shape
    cout = Wp.shape[1]
    hc = h // NCHUNK
    n_global = (h * N_DEV) * w
    xt = jnp.transpose(x, (0, 1, 3, 2))

    def stats_body(x_ref, st_ref, stats_ref, send_sems, recv_sems):
        my = lax.axis_index("i")

        barrier_sem = pltpu.get_barrier_semaphore()
        for d in (1, 2, 3):
            pl.semaphore_signal(
                barrier_sem,
                inc=1,
                device_id=((my + d) % N_DEV,),
                device_id_type=pl.DeviceIdType.MESH,
            )

        for bi in range(b):
            v = x_ref[bi]
            s_cw = jnp.sum(v, axis=0)
            sq_cw = jnp.sum(v * v, axis=0)
            stats_ref[N_DEV - 1, 2 * bi] = jnp.sum(s_cw, axis=1, keepdims=True)
            stats_ref[N_DEV - 1, 2 * bi + 1] = jnp.sum(
                sq_cw, axis=1, keepdims=True
            )

        pl.semaphore_wait(barrier_sem, N_DEV - 1)
        sends = []
        for d in (1, 2, 3):
            rdma = pltpu.make_async_remote_copy(
                src_ref=stats_ref.at[N_DEV - 1],
                dst_ref=stats_ref.at[d - 1],
                send_sem=send_sems.at[d - 1],
                recv_sem=recv_sems.at[d - 1],
                device_id=((my + d) % N_DEV,),
                device_id_type=pl.DeviceIdType.MESH,
            )
            rdma.start()
            sends.append(rdma)
        for d in (1, 2, 3):
            recv = pltpu.make_async_remote_copy(
                src_ref=stats_ref.at[N_DEV - 1],
                dst_ref=stats_ref.at[d - 1],
                send_sem=send_sems.at[d - 1],
                recv_sem=recv_sems.at[d - 1],
                device_id=((my + d) % N_DEV,),
                device_id_type=pl.DeviceIdType.MESH,
            )
            recv.wait_recv()
        for rdma in sends:
            rdma.wait_send()

        eps = 1e-5
        inv_n = 1.0 / float(n_global)
        for bi in range(b):
            ssum = (
                stats_ref[0, 2 * bi]
                + stats_ref[1, 2 * bi]
                + stats_ref[2, 2 * bi]
                + stats_ref[3, 2 * bi]
            )
            ssq = (
                stats_ref[0, 2 * bi + 1]
                + stats_ref[1, 2 * bi + 1]
                + stats_ref[2, 2 * bi + 1]
                + stats_ref[3, 2 * bi + 1]
            )
            mean = ssum * inv_n
            var = ssq * inv_n - mean * mean
            st_ref[2 * bi] = mean
            st_ref[2 * bi + 1] = lax.rsqrt(var + eps)

    st = pl.pallas_call(
        stats_body,
        out_shape=jax.ShapeDtypeStruct((2 * b, c, 1), jnp.float32),
        in_specs=[pl.BlockSpec(memory_space=pltpu.MemorySpace.VMEM)],
        out_specs=pl.BlockSpec(memory_space=pltpu.MemorySpace.VMEM),
        scratch_shapes=[
            pltpu.VMEM((N_DEV, 2 * b, c, 1), jnp.float32),
            pltpu.SemaphoreType.DMA((N_DEV - 1,)),
            pltpu.SemaphoreType.DMA((N_DEV - 1,)),
        ],
        compiler_params=pltpu.CompilerParams(collective_id=0),
    )(xt)

    def main_body(x_ref, wp_ref, st_ref, out_hbm, outbuf, out_sems):
        wp = wp_ref[:, :]
        out_waits = [None, None]
        k = 0
        for bi in range(b):
            mean = st_ref[2 * bi][None, :, :]
            scale = st_ref[2 * bi + 1][None, :, :]
            for ci in range(NCHUNK):
                slot = k % 2
                if out_waits[slot] is not None:
                    out_waits[slot].wait()
                v = x_ref[bi, pl.ds(ci * hc, hc), :, :]
                hh = (v - mean) * scale
                a = hh * jax.nn.sigmoid(hh)
                res = lax.dot_general(
                    a,
                    wp,
                    dimension_numbers=(((1,), (0,)), ((), ())),
                    preferred_element_type=jnp.float32,
                )
                outbuf[slot] = res.reshape(hc * w, cout)
                cp = pltpu.make_async_copy(
                    outbuf.at[slot],
                    out_hbm.at[bi, pl.ds(ci * hc * w, hc * w), :],
                    out_sems.at[slot],
                )
                cp.start()
                out_waits[slot] = cp
                k += 1
        out_waits[0].wait()
        out_waits[1].wait()

    out = pl.pallas_call(
        main_body,
        out_shape=jax.ShapeDtypeStruct((b, h * w, cout), jnp.float32),
        in_specs=[
            pl.BlockSpec(memory_space=pltpu.MemorySpace.VMEM),
            pl.BlockSpec(memory_space=pltpu.MemorySpace.VMEM),
            pl.BlockSpec(memory_space=pltpu.MemorySpace.VMEM),
        ],
        out_specs=pl.BlockSpec(memory_space=pltpu.MemorySpace.HBM),
        scratch_shapes=[
            pltpu.VMEM((2, hc * w, cout), jnp.float32),
            pltpu.SemaphoreType.DMA((2,)),
        ],
    )(xt, Wp, st)
    return out.reshape(b, h, w, cout)
